# baseline (speedup 1.0000x reference)
import functools

import jax
import jax.numpy as jnp
from jax import lax
from jax.experimental import pallas as pl
from jax.experimental.pallas import tpu as pltpu
from jax.experimental.pallas import tpu_sc as plsc

N = 10000
D = 128
E = 320000
NNEIGH = 32.0

NC = 2
NS = 16
NW = NC * NS
EPW = E // NW
K = 80
NCHUNK = EPW // K
SUBSTRIDE = 624
SUBROWS = 640
ZCHUNKS = SUBROWS // K

BR = 1000
GRID = N // BR

_HIGH = lax.Precision.HIGHEST


def _wid(cid, sid):
    return sid * NC + cid


def _deg_body(dst3, const, degp, idx_v, cz_v, acc_sp):
    cid = lax.axis_index("c")
    sid = lax.axis_index("s")
    wid = _wid(cid, sid)
    base = sid * SUBSTRIDE

    pltpu.sync_copy(const, cz_v)
    for r in range(ZCHUNKS):
        pltpu.sync_copy(cz_v.at[0], acc_sp.at[pl.ds(base + r * K, K)])
    plsc.subcore_barrier()

    pltpu.sync_copy(dst3.at[wid], idx_v)

    def body(j, carry):
        pltpu.sync_copy(cz_v.at[1], acc_sp.at[idx_v.at[j]], add=True)
        return carry

    lax.fori_loop(0, NCHUNK, body, 0)
    plsc.subcore_barrier()

    pltpu.sync_copy(acc_sp.at[pl.ds(base, SUBROWS)],
                    degp.at[cid, pl.ds(base, SUBROWS)])


def _deg_call(dst3, const):
    mesh = plsc.VectorSubcoreMesh(core_axis_name="c", subcore_axis_name="s")
    f = pl.kernel(
        _deg_body,
        out_type=jax.ShapeDtypeStruct((NC, N, 16), jnp.float32),
        mesh=mesh,
        compiler_params=pltpu.CompilerParams(use_tc_tiling_on_sc=False),
        scratch_types=[
            pltpu.VMEM((NCHUNK, K), jnp.int32),
            pltpu.VMEM((2, K, 16), jnp.float32),
            pltpu.VMEM_SHARED((N, 16), jnp.float32),
        ],
    )
    return f(dst3, const)


def _edge_body(y_hbm, src3, dst3, zconst, accp,
               srcv, dstv, rows, acc_sp,
               gsem0, gsem1, dsem0, dsem1, ssem0, ssem1):
    cid = lax.axis_index("c")
    sid = lax.axis_index("s")
    wid = _wid(cid, sid)
    base = sid * SUBSTRIDE

    for r in range(ZCHUNKS):
        pltpu.sync_copy(zconst, acc_sp.at[pl.ds(base + r * K, K)])
    plsc.subcore_barrier()

    pltpu.sync_copy(src3.at[wid], srcv)
    gsems = (gsem0, gsem1)
    dsems = (dsem0, dsem1)

    def issue(j, buf):
        pltpu.async_copy(dst3.at[wid, pl.ds(j, 1)], dstv.at[buf], dsems[buf])
        pltpu.async_copy(y_hbm.at[srcv.at[j]], rows.at[buf], gsems[buf])

    ssems = (ssem0, ssem1)

    def wait_in(j, buf):
        pltpu.make_async_copy(dst3.at[wid, pl.ds(j, 1)], dstv.at[buf],
                              dsems[buf]).wait()
        pltpu.make_async_copy(y_hbm.at[srcv.at[j]], rows.at[buf],
                              gsems[buf]).wait()

    def scatter_start(buf):
        pltpu.async_copy(rows.at[buf], acc_sp.at[dstv.at[buf, 0]],
                         ssems[buf], add=True)

    def scatter_wait(buf):
        pltpu.make_async_copy(rows.at[buf], acc_sp.at[dstv.at[buf, 0]],
                              ssems[buf]).wait()

    issue(0, 0)

    def body(i, carry):
        j = 2 * i
        wait_in(j, 0)

        @pl.when(i > 0)
        def _():
            scatter_wait(1)

        issue(j + 1, 1)
        scatter_start(0)
        wait_in(j + 1, 1)
        scatter_wait(0)
        issue(j + 2, 0)
        scatter_start(1)
        return carry

    lax.fori_loop(0, (NCHUNK - 1) // 2, body, 0)
    wait_in(NCHUNK - 1, 0)
    scatter_wait(1)
    pltpu.sync_copy(rows.at[0], acc_sp.at[dstv.at[0, 0]], add=True)
    plsc.subcore_barrier()

    pltpu.sync_copy(acc_sp.at[pl.ds(base, SUBROWS)],
                    accp.at[cid, pl.ds(base, SUBROWS)])


def _edge_call(y, src3, dst3, zconst):
    mesh = plsc.VectorSubcoreMesh(core_axis_name="c", subcore_axis_name="s")
    f = pl.kernel(
        _edge_body,
        out_type=jax.ShapeDtypeStruct((NC, N, D), jnp.float32),
        mesh=mesh,
        scratch_types=[
            pltpu.VMEM((NCHUNK, K), jnp.int32),
            pltpu.VMEM((2, 1, K), jnp.int32),
            pltpu.VMEM((2, K, D), jnp.float32),
            pltpu.VMEM_SHARED((N, D), jnp.float32),
            pltpu.SemaphoreType.DMA,
            pltpu.SemaphoreType.DMA,
            pltpu.SemaphoreType.DMA,
            pltpu.SemaphoreType.DMA,
            pltpu.SemaphoreType.DMA,
            pltpu.SemaphoreType.DMA,
        ],
    )
    return f(y, src3, dst3, zconst)


def _tca_body(gex_ref, wg_ref, wl_ref, bl_ref, d0_ref, d1_ref,
              y_ref, intra_ref, colsum_ref):
    i = pl.program_id(0)
    x = gex_ref[...]
    deg = d0_ref[0] + d1_ref[0]
    dcol = deg[:, 0:1]
    dinv = jnp.where(dcol > 0.0,
                     lax.rsqrt(jnp.maximum(dcol, 1e-12)),
                     0.0)
    xw = lax.dot_general(x, wg_ref[...], (((1,), (1,)), ((), ())),
                         precision=_HIGH, preferred_element_type=jnp.float32)
    y_ref[...] = xw * dinv
    intra = lax.dot_general(x, wl_ref[...], (((1,), (1,)), ((), ())),
                            precision=_HIGH, preferred_element_type=jnp.float32)
    intra_ref[...] = intra + bl_ref[...]
    ps = jnp.sum(x, axis=0, keepdims=True)

    @pl.when(i == 0)
    def _():
        colsum_ref[...] = ps

    @pl.when(i != 0)
    def _():
        colsum_ref[...] = colsum_ref[...] + ps


def _tca_call(gex, wg, wl, bl2, d0w, d1w):
    return pl.pallas_call(
        _tca_body,
        grid=(GRID,),
        in_specs=[
            pl.BlockSpec((BR, D), lambda i: (i, 0)),
            pl.BlockSpec((D, D), lambda i: (0, 0)),
            pl.BlockSpec((D, D), lambda i: (0, 0)),
            pl.BlockSpec((1, D), lambda i: (0, 0)),
            pl.BlockSpec((1, BR, 16), lambda i: (i, 0, 0)),
            pl.BlockSpec((1, BR, 16), lambda i: (i, 0, 0)),
        ],
        out_specs=[
            pl.BlockSpec((BR, D), lambda i: (i, 0)),
            pl.BlockSpec((BR, D), lambda i: (i, 0)),
            pl.BlockSpec((1, D), lambda i: (0, 0)),
        ],
        out_shape=[
            jax.ShapeDtypeStruct((N, D), jnp.float32),
            jax.ShapeDtypeStruct((N, D), jnp.float32),
            jax.ShapeDtypeStruct((1, D), jnp.float32),
        ],
    )(gex, wg, wl, bl2, d0w, d1w)


def _tcb_body(a0_ref, a1_ref, d0_ref, d1_ref, bg_ref, cs_ref,
              wg_ref, wl_ref, msg_ref, logz_ref):
    i = pl.program_id(0)
    deg = d0_ref[0] + d1_ref[0]
    dcol = deg[:, 0:1]
    dinv = jnp.where(dcol > 0.0,
                     lax.rsqrt(jnp.maximum(dcol, 1e-12)),
                     0.0)
    msg_ref[...] = (a0_ref[...] + a1_ref[...]) * dinv + bg_ref[...]

    @pl.when(i == 0)
    def _():
        n = jnp.float32(N)
        m = cs_ref[...] / n
        wg = wg_ref[...]
        wl = wl_ref[...]
        a = NNEIGH * wg + 2.0 * wl
        gv = lax.dot_general(m, a, (((1,), (1,)), ((), ())),
                             precision=_HIGH,
                             preferred_element_type=jnp.float32)
        g = jnp.sqrt(jnp.sum(gv * gv))
        b2 = wl + (0.5 * NNEIGH) * wg
        mb = lax.dot_general(m, b2, (((1,), (0,)), ((), ())),
                             precision=_HIGH,
                             preferred_element_type=jnp.float32)
        z_mean = -n * jnp.sum(mb * m)
        g_safe = jnp.minimum(g, 20.0)
        z_big = n * (g - jnp.log(jnp.maximum(g, 1e-12)))
        z_small = n * jnp.log(
            (jnp.exp(g_safe) - jnp.exp(-g_safe)) / jnp.maximum(g_safe, 1e-12))
        z = z_mean + jnp.where(g > 20.0, z_big, z_small)
        logz_ref[...] = jnp.full((1, 1), 0.0, jnp.float32) + z


def _tcb_call(a0, a1, d0w, d1w, bg2, colsum, wg, wl):
    return pl.pallas_call(
        _tcb_body,
        grid=(GRID,),
        in_specs=[
            pl.BlockSpec((BR, D), lambda i: (i, 0)),
            pl.BlockSpec((BR, D), lambda i: (i, 0)),
            pl.BlockSpec((1, BR, 16), lambda i: (i, 0, 0)),
            pl.BlockSpec((1, BR, 16), lambda i: (i, 0, 0)),
            pl.BlockSpec((1, D), lambda i: (0, 0)),
            pl.BlockSpec((1, D), lambda i: (0, 0)),
            pl.BlockSpec((D, D), lambda i: (0, 0)),
            pl.BlockSpec((D, D), lambda i: (0, 0)),
        ],
        out_specs=[
            pl.BlockSpec((BR, D), lambda i: (i, 0)),
            pl.BlockSpec((1, 1), lambda i: (0, 0)),
        ],
        out_shape=[
            jax.ShapeDtypeStruct((N, D), jnp.float32),
            jax.ShapeDtypeStruct((1, 1), jnp.float32),
        ],
    )(a0, a1, d0w, d1w, bg2, colsum, wg, wl)


def kernel(edge_index, batch, gex, W_g2g, b_g2g, W_lin, b_lin):
    del batch
    src3 = edge_index[0].reshape(NW, NCHUNK, K)
    dst3 = edge_index[1].reshape(NW, NCHUNK, K)

    const = jnp.concatenate(
        [jnp.zeros((1, K, 16), jnp.float32), jnp.ones((1, K, 16), jnp.float32)],
        axis=0)
    zconst = jnp.zeros((K, D), jnp.float32)
    bl2 = b_lin.reshape(1, D)
    bg2 = b_g2g.reshape(1, D)

    degp = _deg_call(dst3, const)
    d0w = degp[0].reshape(GRID, BR, 16)
    d1w = degp[1].reshape(GRID, BR, 16)

    y, intra, colsum = _tca_call(gex, W_g2g, W_lin, bl2, d0w, d1w)

    accp = _edge_call(y, src3, dst3, zconst)
    acc0, acc1 = accp[0], accp[1]

    msg, logz = _tcb_call(acc0, acc1, d0w, d1w, bg2, colsum, W_g2g, W_lin)
    return (msg, intra, logz)


# speedup vs baseline: 1.0921x; 1.0921x over previous
import functools

import jax
import jax.numpy as jnp
from jax import lax
from jax.experimental import pallas as pl
from jax.experimental.pallas import tpu as pltpu
from jax.experimental.pallas import tpu_sc as plsc

N = 10000
D = 128
E = 320000
NNEIGH = 32.0

NC = 2
NS = 16
NW = NC * NS
EPW = E // NW
K = 80
NCHUNK = EPW // K
SUBSTRIDE = 624
SUBROWS = 640
ZCHUNKS = SUBROWS // K

BR = 2000
GRID = N // BR

_HIGH = lax.Precision.HIGHEST


def _wid(cid, sid):
    return sid * NC + cid


def _deg_body(dst3, const, degp, idx_v, cz_v, acc_sp):
    cid = lax.axis_index("c")
    sid = lax.axis_index("s")
    wid = _wid(cid, sid)
    base = sid * SUBSTRIDE

    pltpu.sync_copy(const, cz_v)
    for r in range(ZCHUNKS):
        pltpu.sync_copy(cz_v.at[0], acc_sp.at[pl.ds(base + r * K, K)])
    plsc.subcore_barrier()

    pltpu.sync_copy(dst3.at[wid], idx_v)

    def body(j, carry):
        pltpu.sync_copy(cz_v.at[1], acc_sp.at[idx_v.at[j]], add=True)
        return carry

    lax.fori_loop(0, NCHUNK, body, 0)
    plsc.subcore_barrier()

    pltpu.sync_copy(acc_sp.at[pl.ds(base, SUBROWS)],
                    degp.at[cid, pl.ds(base, SUBROWS)])


def _deg_call(dst3, const):
    mesh = plsc.VectorSubcoreMesh(core_axis_name="c", subcore_axis_name="s")
    f = pl.kernel(
        _deg_body,
        out_type=jax.ShapeDtypeStruct((NC, N, 16), jnp.float32),
        mesh=mesh,
        compiler_params=pltpu.CompilerParams(use_tc_tiling_on_sc=False),
        scratch_types=[
            pltpu.VMEM((NCHUNK, K), jnp.int32),
            pltpu.VMEM((2, K, 16), jnp.float32),
            pltpu.VMEM_SHARED((N, 16), jnp.float32),
        ],
    )
    return f(dst3, const)


def _edge_body(y_hbm, src3, dst3, zconst, accp,
               srcv, dstv, rows, acc_sp,
               gsem0, gsem1, dsem0, dsem1, ssem0, ssem1):
    cid = lax.axis_index("c")
    sid = lax.axis_index("s")
    wid = _wid(cid, sid)
    base = sid * SUBSTRIDE

    for r in range(ZCHUNKS):
        pltpu.sync_copy(zconst, acc_sp.at[pl.ds(base + r * K, K)])
    plsc.subcore_barrier()

    pltpu.sync_copy(src3.at[wid], srcv)
    gsems = (gsem0, gsem1)
    dsems = (dsem0, dsem1)

    def issue(j, buf):
        pltpu.async_copy(dst3.at[wid, pl.ds(j, 1)], dstv.at[buf], dsems[buf])
        pltpu.async_copy(y_hbm.at[srcv.at[j]], rows.at[buf], gsems[buf])

    ssems = (ssem0, ssem1)

    def wait_in(j, buf):
        pltpu.make_async_copy(dst3.at[wid, pl.ds(j, 1)], dstv.at[buf],
                              dsems[buf]).wait()
        pltpu.make_async_copy(y_hbm.at[srcv.at[j]], rows.at[buf],
                              gsems[buf]).wait()

    def scatter_start(buf):
        pltpu.async_copy(rows.at[buf], acc_sp.at[dstv.at[buf, 0]],
                         ssems[buf], add=True)

    def scatter_wait(buf):
        pltpu.make_async_copy(rows.at[buf], acc_sp.at[dstv.at[buf, 0]],
                              ssems[buf]).wait()

    issue(0, 0)

    def body(i, carry):
        j = 2 * i
        wait_in(j, 0)

        @pl.when(i > 0)
        def _():
            scatter_wait(1)

        issue(j + 1, 1)
        scatter_start(0)
        wait_in(j + 1, 1)
        scatter_wait(0)
        issue(j + 2, 0)
        scatter_start(1)
        return carry

    lax.fori_loop(0, (NCHUNK - 1) // 2, body, 0)
    wait_in(NCHUNK - 1, 0)
    scatter_wait(1)
    pltpu.sync_copy(rows.at[0], acc_sp.at[dstv.at[0, 0]], add=True)
    plsc.subcore_barrier()

    pltpu.sync_copy(acc_sp.at[pl.ds(base, SUBROWS)],
                    accp.at[cid, pl.ds(base, SUBROWS)])


def _edge_call(y, src3, dst3, zconst):
    mesh = plsc.VectorSubcoreMesh(core_axis_name="c", subcore_axis_name="s")
    f = pl.kernel(
        _edge_body,
        out_type=jax.ShapeDtypeStruct((NC, N, D), jnp.float32),
        mesh=mesh,
        scratch_types=[
            pltpu.VMEM((NCHUNK, K), jnp.int32),
            pltpu.VMEM((2, 1, K), jnp.int32),
            pltpu.VMEM((2, K, D), jnp.float32),
            pltpu.VMEM_SHARED((N, D), jnp.float32),
            pltpu.SemaphoreType.DMA,
            pltpu.SemaphoreType.DMA,
            pltpu.SemaphoreType.DMA,
            pltpu.SemaphoreType.DMA,
            pltpu.SemaphoreType.DMA,
            pltpu.SemaphoreType.DMA,
        ],
    )
    return f(y, src3, dst3, zconst)


def _tca_body(gex_ref, wg_ref, wl_ref, bl_ref, dp_ref,
              y_ref, intra_ref, colsum_ref):
    i = pl.program_id(0)
    x = gex_ref[...]
    deg = dp_ref[0] + dp_ref[1]
    dcol = deg[:, 0:1]
    dinv = jnp.where(dcol > 0.0,
                     lax.rsqrt(jnp.maximum(dcol, 1e-12)),
                     0.0)
    xw = lax.dot_general(x, wg_ref[...], (((1,), (1,)), ((), ())),
                         precision=_HIGH, preferred_element_type=jnp.float32)
    y_ref[...] = xw * dinv
    intra = lax.dot_general(x, wl_ref[...], (((1,), (1,)), ((), ())),
                            precision=_HIGH, preferred_element_type=jnp.float32)
    intra_ref[...] = intra + bl_ref[...]
    ps = jnp.sum(x, axis=0, keepdims=True)

    @pl.when(i == 0)
    def _():
        colsum_ref[...] = ps

    @pl.when(i != 0)
    def _():
        colsum_ref[...] = colsum_ref[...] + ps


def _tca_call(gex, wg, wl, bl2, degp):
    return pl.pallas_call(
        _tca_body,
        grid=(GRID,),
        in_specs=[
            pl.BlockSpec((BR, D), lambda i: (i, 0)),
            pl.BlockSpec((D, D), lambda i: (0, 0)),
            pl.BlockSpec((D, D), lambda i: (0, 0)),
            pl.BlockSpec((1, D), lambda i: (0, 0)),
            pl.BlockSpec((NC, BR, 16), lambda i: (0, i, 0)),
        ],
        out_specs=[
            pl.BlockSpec((BR, D), lambda i: (i, 0)),
            pl.BlockSpec((BR, D), lambda i: (i, 0)),
            pl.BlockSpec((1, D), lambda i: (0, 0)),
        ],
        out_shape=[
            jax.ShapeDtypeStruct((N, D), jnp.float32),
            jax.ShapeDtypeStruct((N, D), jnp.float32),
            jax.ShapeDtypeStruct((1, D), jnp.float32),
        ],
    )(gex, wg, wl, bl2, degp)


def _tcb_body(ap_ref, dp_ref, bg_ref, cs_ref,
              wg_ref, wl_ref, msg_ref, logz_ref):
    i = pl.program_id(0)
    deg = dp_ref[0] + dp_ref[1]
    dcol = deg[:, 0:1]
    dinv = jnp.where(dcol > 0.0,
                     lax.rsqrt(jnp.maximum(dcol, 1e-12)),
                     0.0)
    msg_ref[...] = (ap_ref[0] + ap_ref[1]) * dinv + bg_ref[...]

    @pl.when(i == 0)
    def _():
        n = jnp.float32(N)
        m = cs_ref[...] / n
        wg = wg_ref[...]
        wl = wl_ref[...]
        a = NNEIGH * wg + 2.0 * wl
        gv = lax.dot_general(m, a, (((1,), (1,)), ((), ())),
                             precision=_HIGH,
                             preferred_element_type=jnp.float32)
        g = jnp.sqrt(jnp.sum(gv * gv))
        b2 = wl + (0.5 * NNEIGH) * wg
        mb = lax.dot_general(m, b2, (((1,), (0,)), ((), ())),
                             precision=_HIGH,
                             preferred_element_type=jnp.float32)
        z_mean = -n * jnp.sum(mb * m)
        g_safe = jnp.minimum(g, 20.0)
        z_big = n * (g - jnp.log(jnp.maximum(g, 1e-12)))
        z_small = n * jnp.log(
            (jnp.exp(g_safe) - jnp.exp(-g_safe)) / jnp.maximum(g_safe, 1e-12))
        z = z_mean + jnp.where(g > 20.0, z_big, z_small)
        logz_ref[...] = jnp.full((1, 1), 0.0, jnp.float32) + z


def _tcb_call(accp, degp, bg2, colsum, wg, wl):
    return pl.pallas_call(
        _tcb_body,
        grid=(GRID,),
        in_specs=[
            pl.BlockSpec((NC, BR, D), lambda i: (0, i, 0)),
            pl.BlockSpec((NC, BR, 16), lambda i: (0, i, 0)),
            pl.BlockSpec((1, D), lambda i: (0, 0)),
            pl.BlockSpec((1, D), lambda i: (0, 0)),
            pl.BlockSpec((D, D), lambda i: (0, 0)),
            pl.BlockSpec((D, D), lambda i: (0, 0)),
        ],
        out_specs=[
            pl.BlockSpec((BR, D), lambda i: (i, 0)),
            pl.BlockSpec((1, 1), lambda i: (0, 0)),
        ],
        out_shape=[
            jax.ShapeDtypeStruct((N, D), jnp.float32),
            jax.ShapeDtypeStruct((1, 1), jnp.float32),
        ],
    )(accp, degp, bg2, colsum, wg, wl)


def kernel(edge_index, batch, gex, W_g2g, b_g2g, W_lin, b_lin):
    del batch
    src3 = edge_index[0].reshape(NW, NCHUNK, K)
    dst3 = edge_index[1].reshape(NW, NCHUNK, K)

    const = jnp.concatenate(
        [jnp.zeros((1, K, 16), jnp.float32), jnp.ones((1, K, 16), jnp.float32)],
        axis=0)
    zconst = jnp.zeros((K, D), jnp.float32)
    bl2 = b_lin.reshape(1, D)
    bg2 = b_g2g.reshape(1, D)

    degp = _deg_call(dst3, const)

    y, intra, colsum = _tca_call(gex, W_g2g, W_lin, bl2, degp)

    accp = _edge_call(y, src3, dst3, zconst)

    msg, logz = _tcb_call(accp, degp, bg2, colsum, W_g2g, W_lin)
    return (msg, intra, logz)


# speedup vs baseline: 1.1622x; 1.0642x over previous
import functools

import jax
import jax.numpy as jnp
from jax import lax
from jax.experimental import pallas as pl
from jax.experimental.pallas import tpu as pltpu
from jax.experimental.pallas import tpu_sc as plsc

N = 10000
D = 128
E = 320000
NNEIGH = 32.0

NC = 2
NS = 16
NW = NC * NS
EPW = E // NW
K = 80
NCHUNK = EPW // K
SUBSTRIDE = 624
SUBROWS = 640
ZCHUNKS = SUBROWS // K

BR = 2000
GRID = N // BR

_HIGH = lax.Precision.HIGHEST


def _wid(cid, sid):
    return sid * NC + cid


def _deg_body(dst3, const, degp, idx_v, cz_v, acc_sp):
    cid = lax.axis_index("c")
    sid = lax.axis_index("s")
    wid = _wid(cid, sid)
    base = sid * SUBSTRIDE

    pltpu.sync_copy(const, cz_v)
    for r in range(ZCHUNKS):
        pltpu.sync_copy(cz_v.at[0], acc_sp.at[pl.ds(base + r * K, K)])
    plsc.subcore_barrier()

    pltpu.sync_copy(dst3.at[wid], idx_v)

    def body(j, carry):
        pltpu.sync_copy(cz_v.at[1], acc_sp.at[idx_v.at[j]], add=True)
        return carry

    lax.fori_loop(0, NCHUNK, body, 0)
    plsc.subcore_barrier()

    pltpu.sync_copy(acc_sp.at[pl.ds(base, SUBROWS)],
                    degp.at[cid, pl.ds(base, SUBROWS)])


def _deg_call(dst3, const):
    mesh = plsc.VectorSubcoreMesh(core_axis_name="c", subcore_axis_name="s")
    f = pl.kernel(
        _deg_body,
        out_type=jax.ShapeDtypeStruct((NC, N, 16), jnp.float32),
        mesh=mesh,
        compiler_params=pltpu.CompilerParams(use_tc_tiling_on_sc=False),
        scratch_types=[
            pltpu.VMEM((NCHUNK, K), jnp.int32),
            pltpu.VMEM((2, K, 16), jnp.float32),
            pltpu.VMEM_SHARED((N, 16), jnp.float32),
        ],
    )
    return f(dst3, const)


def _edge_body(y_hbm, src3, dst3, zconst, accp,
               srcv, dstv, rows, acc_sp,
               gsem0, gsem1, dsem0, dsem1, ssem0, ssem1):
    cid = lax.axis_index("c")
    sid = lax.axis_index("s")
    wid = _wid(cid, sid)
    base = sid * SUBSTRIDE

    for r in range(ZCHUNKS):
        pltpu.sync_copy(zconst, acc_sp.at[pl.ds(base + r * K, K)])
    plsc.subcore_barrier()

    pltpu.sync_copy(src3.at[wid], srcv)
    gsems = (gsem0, gsem1)
    dsems = (dsem0, dsem1)

    def issue(j, buf):
        pltpu.async_copy(dst3.at[wid, pl.ds(j, 1)], dstv.at[buf], dsems[buf])
        pltpu.async_copy(y_hbm.at[srcv.at[j]], rows.at[buf], gsems[buf])

    ssems = (ssem0, ssem1)

    def wait_in(j, buf):
        pltpu.make_async_copy(dst3.at[wid, pl.ds(j, 1)], dstv.at[buf],
                              dsems[buf]).wait()
        pltpu.make_async_copy(y_hbm.at[srcv.at[j]], rows.at[buf],
                              gsems[buf]).wait()

    def scatter_start(buf):
        pltpu.async_copy(rows.at[buf], acc_sp.at[dstv.at[buf, 0]],
                         ssems[buf], add=True)

    def scatter_wait(buf):
        pltpu.make_async_copy(rows.at[buf], acc_sp.at[dstv.at[buf, 0]],
                              ssems[buf]).wait()

    issue(0, 0)

    def body(i, carry):
        j = 2 * i
        wait_in(j, 0)

        @pl.when(i > 0)
        def _():
            scatter_wait(1)

        issue(j + 1, 1)
        scatter_start(0)
        wait_in(j + 1, 1)
        scatter_wait(0)
        issue(j + 2, 0)
        scatter_start(1)
        return carry

    lax.fori_loop(0, (NCHUNK - 1) // 2, body, 0)
    wait_in(NCHUNK - 1, 0)
    scatter_wait(1)
    pltpu.sync_copy(rows.at[0], acc_sp.at[dstv.at[0, 0]], add=True)
    plsc.subcore_barrier()

    pltpu.sync_copy(acc_sp.at[pl.ds(base, SUBROWS)],
                    accp.at[cid, pl.ds(base, SUBROWS)])


def _edge_call(y, src3, dst3, zconst):
    mesh = plsc.VectorSubcoreMesh(core_axis_name="c", subcore_axis_name="s")
    f = pl.kernel(
        _edge_body,
        out_type=jax.ShapeDtypeStruct((NC, N, D), jnp.bfloat16),
        mesh=mesh,
        compiler_params=pltpu.CompilerParams(use_tc_tiling_on_sc=False),
        scratch_types=[
            pltpu.VMEM((NCHUNK, K), jnp.int32),
            pltpu.VMEM((2, 1, K), jnp.int32),
            pltpu.VMEM((2, K, D), jnp.bfloat16),
            pltpu.VMEM_SHARED((N, D), jnp.bfloat16),
            pltpu.SemaphoreType.DMA,
            pltpu.SemaphoreType.DMA,
            pltpu.SemaphoreType.DMA,
            pltpu.SemaphoreType.DMA,
            pltpu.SemaphoreType.DMA,
            pltpu.SemaphoreType.DMA,
        ],
    )
    return f(y, src3, dst3, zconst)


def _tca_body(gex_ref, wg_ref, wl_ref, bl_ref, dp_ref,
              y_ref, intra_ref, colsum_ref):
    i = pl.program_id(0)
    x = gex_ref[...]
    deg = dp_ref[0] + dp_ref[1]
    dcol = deg[:, 0:1]
    dinv = jnp.where(dcol > 0.0,
                     lax.rsqrt(jnp.maximum(dcol, 1e-12)),
                     0.0)
    xw = lax.dot_general(x, wg_ref[...], (((1,), (1,)), ((), ())),
                         precision=_HIGH, preferred_element_type=jnp.float32)
    y_ref[...] = (xw * dinv).astype(jnp.bfloat16)
    intra = lax.dot_general(x, wl_ref[...], (((1,), (1,)), ((), ())),
                            precision=_HIGH, preferred_element_type=jnp.float32)
    intra_ref[...] = intra + bl_ref[...]
    ps = jnp.sum(x, axis=0, keepdims=True)

    @pl.when(i == 0)
    def _():
        colsum_ref[...] = ps

    @pl.when(i != 0)
    def _():
        colsum_ref[...] = colsum_ref[...] + ps


def _tca_call(gex, wg, wl, bl2, degp):
    return pl.pallas_call(
        _tca_body,
        grid=(GRID,),
        in_specs=[
            pl.BlockSpec((BR, D), lambda i: (i, 0)),
            pl.BlockSpec((D, D), lambda i: (0, 0)),
            pl.BlockSpec((D, D), lambda i: (0, 0)),
            pl.BlockSpec((1, D), lambda i: (0, 0)),
            pl.BlockSpec((NC, BR, 16), lambda i: (0, i, 0)),
        ],
        out_specs=[
            pl.BlockSpec((BR, D), lambda i: (i, 0)),
            pl.BlockSpec((BR, D), lambda i: (i, 0)),
            pl.BlockSpec((1, D), lambda i: (0, 0)),
        ],
        out_shape=[
            jax.ShapeDtypeStruct((N, D), jnp.bfloat16),
            jax.ShapeDtypeStruct((N, D), jnp.float32),
            jax.ShapeDtypeStruct((1, D), jnp.float32),
        ],
    )(gex, wg, wl, bl2, degp)


def _tcb_body(ap_ref, dp_ref, bg_ref, cs_ref,
              wg_ref, wl_ref, msg_ref, logz_ref):
    i = pl.program_id(0)
    deg = dp_ref[0] + dp_ref[1]
    dcol = deg[:, 0:1]
    dinv = jnp.where(dcol > 0.0,
                     lax.rsqrt(jnp.maximum(dcol, 1e-12)),
                     0.0)
    acc = ap_ref[0].astype(jnp.float32) + ap_ref[1].astype(jnp.float32)
    msg_ref[...] = acc * dinv + bg_ref[...]

    @pl.when(i == 0)
    def _():
        n = jnp.float32(N)
        m = cs_ref[...] / n
        wg = wg_ref[...]
        wl = wl_ref[...]
        a = NNEIGH * wg + 2.0 * wl
        gv = lax.dot_general(m, a, (((1,), (1,)), ((), ())),
                             precision=_HIGH,
                             preferred_element_type=jnp.float32)
        g = jnp.sqrt(jnp.sum(gv * gv))
        b2 = wl + (0.5 * NNEIGH) * wg
        mb = lax.dot_general(m, b2, (((1,), (0,)), ((), ())),
                             precision=_HIGH,
                             preferred_element_type=jnp.float32)
        z_mean = -n * jnp.sum(mb * m)
        g_safe = jnp.minimum(g, 20.0)
        z_big = n * (g - jnp.log(jnp.maximum(g, 1e-12)))
        z_small = n * jnp.log(
            (jnp.exp(g_safe) - jnp.exp(-g_safe)) / jnp.maximum(g_safe, 1e-12))
        z = z_mean + jnp.where(g > 20.0, z_big, z_small)
        logz_ref[...] = jnp.full((1, 1), 0.0, jnp.float32) + z


def _tcb_call(accp, degp, bg2, colsum, wg, wl):
    return pl.pallas_call(
        _tcb_body,
        grid=(GRID,),
        in_specs=[
            pl.BlockSpec((NC, BR, D), lambda i: (0, i, 0)),
            pl.BlockSpec((NC, BR, 16), lambda i: (0, i, 0)),
            pl.BlockSpec((1, D), lambda i: (0, 0)),
            pl.BlockSpec((1, D), lambda i: (0, 0)),
            pl.BlockSpec((D, D), lambda i: (0, 0)),
            pl.BlockSpec((D, D), lambda i: (0, 0)),
        ],
        out_specs=[
            pl.BlockSpec((BR, D), lambda i: (i, 0)),
            pl.BlockSpec((1, 1), lambda i: (0, 0)),
        ],
        out_shape=[
            jax.ShapeDtypeStruct((N, D), jnp.float32),
            jax.ShapeDtypeStruct((1, 1), jnp.float32),
        ],
    )(accp, degp, bg2, colsum, wg, wl)


def kernel(edge_index, batch, gex, W_g2g, b_g2g, W_lin, b_lin):
    del batch
    src3 = edge_index[0].reshape(NW, NCHUNK, K)
    dst3 = edge_index[1].reshape(NW, NCHUNK, K)

    const = jnp.concatenate(
        [jnp.zeros((1, K, 16), jnp.float32), jnp.ones((1, K, 16), jnp.float32)],
        axis=0)
    zconst = jnp.zeros((K, D), jnp.bfloat16)
    bl2 = b_lin.reshape(1, D)
    bg2 = b_g2g.reshape(1, D)

    degp = _deg_call(dst3, const)

    y, intra, colsum = _tca_call(gex, W_g2g, W_lin, bl2, degp)

    accp = _edge_call(y, src3, dst3, zconst)

    msg, logz = _tcb_call(accp, degp, bg2, colsum, W_g2g, W_lin)
    return (msg, intra, logz)


# speedup vs baseline: 1.6240x; 1.3973x over previous
import functools

import jax
import jax.numpy as jnp
from jax import lax
from jax.experimental import pallas as pl
from jax.experimental.pallas import tpu as pltpu
from jax.experimental.pallas import tpu_sc as plsc

N = 10000
D = 128
E = 320000
NNEIGH = 32.0

NC = 2
NS = 16
NW = NC * NS
EPW = E // NW
K = 125
NCHUNK = EPW // K
NBUF = 4
SUBSTRIDE = 624
SUBROWS = 640

BR = 2000
GRID = N // BR

_HIGH = lax.Precision.HIGHEST


def _wid(cid, sid):
    return sid * NC + cid


def _deg_body(dst3, dzero, dones, degp, idx_v, ones_v, acc_sp, ssem):
    cid = lax.axis_index("c")
    sid = lax.axis_index("s")
    wid = _wid(cid, sid)
    base = sid * SUBSTRIDE

    pltpu.sync_copy(dones, ones_v)
    pltpu.sync_copy(dzero, acc_sp.at[pl.ds(base, SUBROWS)])
    plsc.subcore_barrier()

    pltpu.sync_copy(dst3.at[wid], idx_v)

    def scat(j):
        pltpu.async_copy(ones_v, acc_sp.at[idx_v.at[j]], ssem, add=True)

    def scat_wait(j):
        pltpu.make_async_copy(ones_v, acc_sp.at[idx_v.at[j]], ssem).wait()

    for j in range(NBUF - 1):
        scat(j)

    def body(j, carry):
        scat(j + NBUF - 1)
        scat_wait(j)
        return carry

    lax.fori_loop(0, NCHUNK - NBUF + 1, body, 0)
    for j in range(NCHUNK - NBUF + 1, NCHUNK):
        scat_wait(j)
    plsc.subcore_barrier()

    pltpu.sync_copy(acc_sp.at[pl.ds(base, SUBROWS)],
                    degp.at[cid, pl.ds(base, SUBROWS)])


def _deg_call(dst3, dzero, dones):
    mesh = plsc.VectorSubcoreMesh(core_axis_name="c", subcore_axis_name="s")
    f = pl.kernel(
        _deg_body,
        out_type=jax.ShapeDtypeStruct((NC, N, 16), jnp.float32),
        mesh=mesh,
        compiler_params=pltpu.CompilerParams(use_tc_tiling_on_sc=False),
        scratch_types=[
            pltpu.VMEM((NCHUNK, K), jnp.int32),
            pltpu.VMEM((K, 16), jnp.float32),
            pltpu.VMEM_SHARED((N, 16), jnp.float32),
            pltpu.SemaphoreType.DMA,
        ],
    )
    return f(dst3, dzero, dones)


def _edge_body(y_hbm, src3, dst3, zconst, accp,
               srcv, dstv, rows, acc_sp, gsems, dsems, ssems):
    cid = lax.axis_index("c")
    sid = lax.axis_index("s")
    wid = _wid(cid, sid)
    base = sid * SUBSTRIDE

    pltpu.sync_copy(zconst, acc_sp.at[pl.ds(base, SUBROWS)])
    plsc.subcore_barrier()

    pltpu.sync_copy(src3.at[wid], srcv)

    def issue(j, b):
        pltpu.async_copy(dst3.at[wid, pl.ds(j, 1)], dstv.at[b], dsems.at[b])
        pltpu.async_copy(y_hbm.at[srcv.at[j]], rows.at[b], gsems.at[b])

    def wait_in(j, b):
        pltpu.make_async_copy(dst3.at[wid, pl.ds(j, 1)], dstv.at[b],
                              dsems.at[b]).wait()
        pltpu.make_async_copy(y_hbm.at[srcv.at[j]], rows.at[b],
                              gsems.at[b]).wait()

    def scatter_start(b):
        pltpu.async_copy(rows.at[b], acc_sp.at[dstv.at[b, 0]],
                         ssems.at[b], add=True)

    def scatter_wait(b):
        pltpu.make_async_copy(rows.at[b], acc_sp.at[dstv.at[b, 0]],
                              ssems.at[b]).wait()

    for j in range(NBUF - 1):
        issue(j, j)

    def body(i, carry):
        for b in range(NBUF):
            j = NBUF * i + b
            nb = (b + NBUF - 1) % NBUF
            wait_in(j, b)
            scatter_start(b)

            if b == 0:
                @pl.when(j >= 1)
                def _():
                    scatter_wait(nb)
            else:
                scatter_wait(nb)

            @pl.when(j + NBUF - 1 < NCHUNK)
            def _():
                issue(j + NBUF - 1, nb)

        return carry

    lax.fori_loop(0, NCHUNK // NBUF, body, 0)
    scatter_wait((NCHUNK - 1) % NBUF)
    plsc.subcore_barrier()

    pltpu.sync_copy(acc_sp.at[pl.ds(base, SUBROWS)],
                    accp.at[cid, pl.ds(base, SUBROWS)])


def _edge_call(y, src3, dst3, zconst):
    mesh = plsc.VectorSubcoreMesh(core_axis_name="c", subcore_axis_name="s")
    f = pl.kernel(
        _edge_body,
        out_type=jax.ShapeDtypeStruct((NC, N, D), jnp.bfloat16),
        mesh=mesh,
        compiler_params=pltpu.CompilerParams(use_tc_tiling_on_sc=False),
        scratch_types=[
            pltpu.VMEM((NCHUNK, K), jnp.int32),
            pltpu.VMEM((NBUF, 1, K), jnp.int32),
            pltpu.VMEM((NBUF, K, D), jnp.bfloat16),
            pltpu.VMEM_SHARED((N, D), jnp.bfloat16),
            pltpu.SemaphoreType.DMA((NBUF,)),
            pltpu.SemaphoreType.DMA((NBUF,)),
            pltpu.SemaphoreType.DMA((NBUF,)),
        ],
    )
    return f(y, src3, dst3, zconst)


def _tca_body(gex_ref, wg_ref, wl_ref, bl_ref, dp_ref,
              y_ref, intra_ref, colsum_ref):
    i = pl.program_id(0)
    x = gex_ref[...]
    deg = dp_ref[0] + dp_ref[1]
    dcol = deg[:, 0:1]
    dinv = jnp.where(dcol > 0.0,
                     lax.rsqrt(jnp.maximum(dcol, 1e-12)),
                     0.0)
    xw = lax.dot_general(x, wg_ref[...], (((1,), (1,)), ((), ())),
                         precision=_HIGH, preferred_element_type=jnp.float32)
    y_ref[...] = (xw * dinv).astype(jnp.bfloat16)
    intra = lax.dot_general(x, wl_ref[...], (((1,), (1,)), ((), ())),
                            precision=_HIGH, preferred_element_type=jnp.float32)
    intra_ref[...] = intra + bl_ref[...]
    ps = jnp.sum(x, axis=0, keepdims=True)

    @pl.when(i == 0)
    def _():
        colsum_ref[...] = ps

    @pl.when(i != 0)
    def _():
        colsum_ref[...] = colsum_ref[...] + ps


def _tca_call(gex, wg, wl, bl2, degp):
    return pl.pallas_call(
        _tca_body,
        grid=(GRID,),
        in_specs=[
            pl.BlockSpec((BR, D), lambda i: (i, 0)),
            pl.BlockSpec((D, D), lambda i: (0, 0)),
            pl.BlockSpec((D, D), lambda i: (0, 0)),
            pl.BlockSpec((1, D), lambda i: (0, 0)),
            pl.BlockSpec((NC, BR, 16), lambda i: (0, i, 0)),
        ],
        out_specs=[
            pl.BlockSpec((BR, D), lambda i: (i, 0)),
            pl.BlockSpec((BR, D), lambda i: (i, 0)),
            pl.BlockSpec((1, D), lambda i: (0, 0)),
        ],
        out_shape=[
            jax.ShapeDtypeStruct((N, D), jnp.bfloat16),
            jax.ShapeDtypeStruct((N, D), jnp.float32),
            jax.ShapeDtypeStruct((1, D), jnp.float32),
        ],
    )(gex, wg, wl, bl2, degp)


def _tcb_body(ap_ref, dp_ref, bg_ref, cs_ref,
              wg_ref, wl_ref, msg_ref, logz_ref):
    i = pl.program_id(0)
    deg = dp_ref[0] + dp_ref[1]
    dcol = deg[:, 0:1]
    dinv = jnp.where(dcol > 0.0,
                     lax.rsqrt(jnp.maximum(dcol, 1e-12)),
                     0.0)
    acc = ap_ref[0].astype(jnp.float32) + ap_ref[1].astype(jnp.float32)
    msg_ref[...] = acc * dinv + bg_ref[...]

    @pl.when(i == 0)
    def _():
        n = jnp.float32(N)
        m = cs_ref[...] / n
        wg = wg_ref[...]
        wl = wl_ref[...]
        a = NNEIGH * wg + 2.0 * wl
        gv = lax.dot_general(m, a, (((1,), (1,)), ((), ())),
                             precision=_HIGH,
                             preferred_element_type=jnp.float32)
        g = jnp.sqrt(jnp.sum(gv * gv))
        b2 = wl + (0.5 * NNEIGH) * wg
        mb = lax.dot_general(m, b2, (((1,), (0,)), ((), ())),
                             precision=_HIGH,
                             preferred_element_type=jnp.float32)
        z_mean = -n * jnp.sum(mb * m)
        g_safe = jnp.minimum(g, 20.0)
        z_big = n * (g - jnp.log(jnp.maximum(g, 1e-12)))
        z_small = n * jnp.log(
            (jnp.exp(g_safe) - jnp.exp(-g_safe)) / jnp.maximum(g_safe, 1e-12))
        z = z_mean + jnp.where(g > 20.0, z_big, z_small)
        logz_ref[...] = jnp.full((1, 1), 0.0, jnp.float32) + z


def _tcb_call(accp, degp, bg2, colsum, wg, wl):
    return pl.pallas_call(
        _tcb_body,
        grid=(GRID,),
        in_specs=[
            pl.BlockSpec((NC, BR, D), lambda i: (0, i, 0)),
            pl.BlockSpec((NC, BR, 16), lambda i: (0, i, 0)),
            pl.BlockSpec((1, D), lambda i: (0, 0)),
            pl.BlockSpec((1, D), lambda i: (0, 0)),
            pl.BlockSpec((D, D), lambda i: (0, 0)),
            pl.BlockSpec((D, D), lambda i: (0, 0)),
        ],
        out_specs=[
            pl.BlockSpec((BR, D), lambda i: (i, 0)),
            pl.BlockSpec((1, 1), lambda i: (0, 0)),
        ],
        out_shape=[
            jax.ShapeDtypeStruct((N, D), jnp.float32),
            jax.ShapeDtypeStruct((1, 1), jnp.float32),
        ],
    )(accp, degp, bg2, colsum, wg, wl)


def kernel(edge_index, batch, gex, W_g2g, b_g2g, W_lin, b_lin):
    del batch
    src3 = edge_index[0].reshape(NW, NCHUNK, K)
    dst3 = edge_index[1].reshape(NW, NCHUNK, K)

    dzero = jnp.zeros((SUBROWS, 16), jnp.float32)
    dones = jnp.ones((K, 16), jnp.float32)
    zconst = jnp.zeros((SUBROWS, D), jnp.bfloat16)
    bl2 = b_lin.reshape(1, D)
    bg2 = b_g2g.reshape(1, D)

    degp = _deg_call(dst3, dzero, dones)

    y, intra, colsum = _tca_call(gex, W_g2g, W_lin, bl2, degp)

    accp = _edge_call(y, src3, dst3, zconst)

    msg, logz = _tcb_call(accp, degp, bg2, colsum, W_g2g, W_lin)
    return (msg, intra, logz)
